# baseline (speedup 1.0000x reference)
import functools
import math

import jax
import jax.numpy as jnp
from jax.experimental import pallas as pl
from jax.experimental.pallas import tpu as pltpu

_BLOCK = 8192
_TOPK = 8
_NEG_INF = float("-inf")
_IMAX = jnp.iinfo(jnp.int32).max


def _group_sum4(v, bit0, bit1):
    d = v.shape[1]
    p1 = jnp.where(bit0, pltpu.roll(v, 1, axis=1),
                   pltpu.roll(v, d - 1, axis=1))
    s1 = v + p1
    p2 = jnp.where(bit1, pltpu.roll(s1, 2, axis=1),
                   pltpu.roll(s1, d - 2, axis=1))
    return s1 + p2


def _scan_body(x_ref, vk_ref, topv_ref, topi_ref, *, n_keys, block2):
    b = pl.program_id(0)
    d = x_ref.shape[1]
    q = x_ref.shape[0]

    x = x_ref[...]
    lane_q = jax.lax.broadcasted_iota(jnp.int32, (q, d), 1)
    xn = jnp.sqrt(_group_sum4(x * x, lane_q % 2 == 1,
                              (lane_q // 2) % 2 == 1) + 1e-8)
    qn = (x / xn) * 0.5
    zq = jnp.zeros_like(qn)
    qlo = jnp.concatenate([qn, zq], axis=1)
    qhi = jnp.concatenate([zq, qn], axis=1)

    kb = vk_ref[...]
    lane_k = jax.lax.broadcasted_iota(jnp.int32, (block2, 2 * d), 1)
    kn = jnp.sqrt(_group_sum4(kb * kb, lane_k % 2 == 1,
                              (lane_k // 2) % 2 == 1) + 1e-8)
    knb = kb / kn

    s_even = jax.lax.dot_general(qlo, knb, (((1,), (1,)), ((), ())),
                                 preferred_element_type=jnp.float32)
    s_odd = jax.lax.dot_general(qhi, knb, (((1,), (1,)), ((), ())),
                                preferred_element_type=jnp.float32)

    base = b * 2 * block2
    gi_even = jax.lax.broadcasted_iota(jnp.int32, (q, block2), 1) * 2 + base
    gi_odd = gi_even + 1
    s_even = jnp.where(gi_even < n_keys, s_even, _NEG_INF)
    s_odd = jnp.where(gi_odd < n_keys, s_odd, _NEG_INF)
    gi_even = jnp.where(gi_even < n_keys, gi_even, _IMAX)
    gi_odd = jnp.where(gi_odd < n_keys, gi_odd, _IMAX)

    @pl.when(b == 0)
    def _init():
        topv_ref[...] = jnp.full((q, _TOPK), _NEG_INF, jnp.float32)
        topi_ref[...] = jnp.full((q, _TOPK), _IMAX, jnp.int32)

    m1 = jnp.maximum(jnp.max(s_even, axis=1), jnp.max(s_odd, axis=1))
    do_merge = jnp.any(m1 > topv_ref[:, _TOPK - 1])

    @pl.when(do_merge)
    def _merge():
        cs = jnp.concatenate([s_even, s_odd, topv_ref[...]], axis=1)
        ci_all = jnp.concatenate([gi_even, gi_odd, topi_ref[...]], axis=1)

        lane = jax.lax.broadcasted_iota(jnp.int32, (q, _TOPK), 1)
        rv = jnp.full((q, _TOPK), _NEG_INF, jnp.float32)
        ri_out = jnp.full((q, _TOPK), _IMAX, jnp.int32)
        for j in range(_TOPK):
            m = jnp.max(cs, axis=1)
            sel = cs == m[:, None]
            cand = jnp.where(sel, ci_all, _IMAX)
            pick = jnp.min(cand, axis=1)
            chosen = sel & (ci_all == pick[:, None])
            rv = jnp.where(lane == j, m[:, None], rv)
            ri_out = jnp.where(lane == j, pick[:, None], ri_out)
            cs = jnp.where(chosen, _NEG_INF, cs)

        topv_ref[...] = rv
        topi_ref[...] = ri_out


def _readout_body(idx_ref, tv_ref, *row_and_out_refs):
    row_refs = row_and_out_refs[:_TOPK]
    out_ref = row_and_out_refs[_TOPK]
    tv = tv_ref[...]
    m = jnp.max(tv)
    e = jnp.exp((tv - m) * 10.0)
    w = e / jnp.sum(e)
    acc = jnp.zeros(out_ref.shape, jnp.float32)
    for j in range(_TOPK):
        acc = acc + w[:, :, j:j + 1] * row_refs[j][...]
    out_ref[...] = acc


def kernel(x, vault_keys):
    n_keys, d = vault_keys.shape
    q = x.shape[0]
    block2 = _BLOCK // 2
    vk_packed = vault_keys.reshape(n_keys // 2, 2 * d)
    nb = math.ceil((n_keys // 2) / block2)

    topv, topi = pl.pallas_call(
        functools.partial(_scan_body, n_keys=n_keys, block2=block2),
        grid=(nb,),
        in_specs=[
            pl.BlockSpec((q, d), lambda b: (0, 0)),
            pl.BlockSpec((block2, 2 * d), lambda b: (b, 0)),
        ],
        out_specs=[
            pl.BlockSpec((q, _TOPK), lambda b: (0, 0)),
            pl.BlockSpec((q, _TOPK), lambda b: (0, 0)),
        ],
        out_shape=[
            jax.ShapeDtypeStruct((q, _TOPK), jnp.float32),
            jax.ShapeDtypeStruct((q, _TOPK), jnp.int32),
        ],
    )(x, vk_packed)

    v3 = vault_keys.reshape(n_keys, 1, d)
    tv3 = topv.reshape(q, 1, _TOPK)
    idx_flat = topi.reshape(-1)

    def _row_spec(j):
        return pl.BlockSpec((1, 1, d),
                            lambda qi, idx, j=j: (idx[qi * _TOPK + j], 0, 0))

    out3 = pl.pallas_call(
        _readout_body,
        grid_spec=pltpu.PrefetchScalarGridSpec(
            num_scalar_prefetch=1,
            grid=(q,),
            in_specs=[pl.BlockSpec((1, 1, _TOPK), lambda qi, idx: (qi, 0, 0))]
            + [_row_spec(j) for j in range(_TOPK)],
            out_specs=pl.BlockSpec((1, 1, d), lambda qi, idx: (qi, 0, 0)),
        ),
        out_shape=jax.ShapeDtypeStruct((q, 1, d), jnp.float32),
    )(idx_flat, tv3, *([v3] * _TOPK))

    return out3.reshape(q, d)


# speedup vs baseline: 1.0433x; 1.0433x over previous
import functools
import math

import jax
import jax.numpy as jnp
from jax.experimental import pallas as pl
from jax.experimental.pallas import tpu as pltpu

_BLOCK = 8000
_TOPK = 8
_NEG_INF = float("-inf")
_IMAX = jnp.iinfo(jnp.int32).max


def _group_sum4(v, bit0, bit1):
    d = v.shape[1]
    p1 = jnp.where(bit0, pltpu.roll(v, 1, axis=1),
                   pltpu.roll(v, d - 1, axis=1))
    s1 = v + p1
    p2 = jnp.where(bit1, pltpu.roll(s1, 2, axis=1),
                   pltpu.roll(s1, d - 2, axis=1))
    return s1 + p2


def _scan_body(x_ref, vk_ref, topv_ref, topi_ref, *, block, n_keys):
    b = pl.program_id(0)
    d = x_ref.shape[1]
    q = x_ref.shape[0]

    x = x_ref[...]
    lane_q = jax.lax.broadcasted_iota(jnp.int32, (q, d), 1)
    xn = jnp.sqrt(_group_sum4(x * x, lane_q % 2 == 1,
                              (lane_q // 2) % 2 == 1) + 1e-8)
    qn = (x / xn) * 0.5

    kb = vk_ref[...]
    g0 = jax.lax.broadcasted_iota(jnp.int32, (d, d), 0) // 4
    g1 = jax.lax.broadcasted_iota(jnp.int32, (d, d), 1) // 4
    mask = (g0 == g1).astype(jnp.float32)
    gs = jax.lax.dot_general(kb * kb, mask, (((1,), (0,)), ((), ())),
                             preferred_element_type=jnp.float32,
                             precision=jax.lax.Precision.HIGHEST)
    knb = kb / jnp.sqrt(gs + 1e-8)

    s = jax.lax.dot_general(qn, knb, (((1,), (1,)), ((), ())),
                            preferred_element_type=jnp.float32)

    gi = jax.lax.broadcasted_iota(jnp.int32, (q, block), 1) + b * block
    if n_keys % block != 0:
        s = jnp.where(gi < n_keys, s, _NEG_INF)
        gi = jnp.where(gi < n_keys, gi, _IMAX)

    @pl.when(b == 0)
    def _init():
        topv_ref[...] = jnp.full((q, _TOPK), _NEG_INF, jnp.float32)
        topi_ref[...] = jnp.full((q, _TOPK), _IMAX, jnp.int32)

    do_merge = jnp.any(jnp.max(s, axis=1) > topv_ref[:, _TOPK - 1])

    @pl.when(do_merge)
    def _merge():
        cs = jnp.concatenate([s, topv_ref[...]], axis=1)
        ci_all = jnp.concatenate([gi, topi_ref[...]], axis=1)

        lane = jax.lax.broadcasted_iota(jnp.int32, (q, _TOPK), 1)
        rv = jnp.full((q, _TOPK), _NEG_INF, jnp.float32)
        ri_out = jnp.full((q, _TOPK), _IMAX, jnp.int32)
        for j in range(_TOPK):
            m = jnp.max(cs, axis=1)
            sel = cs == m[:, None]
            cand = jnp.where(sel, ci_all, _IMAX)
            pick = jnp.min(cand, axis=1)
            chosen = sel & (ci_all == pick[:, None])
            rv = jnp.where(lane == j, m[:, None], rv)
            ri_out = jnp.where(lane == j, pick[:, None], ri_out)
            cs = jnp.where(chosen, _NEG_INF, cs)

        topv_ref[...] = rv
        topi_ref[...] = ri_out


def _readout_body(idx_ref, tv_ref, *row_and_out_refs):
    row_refs = row_and_out_refs[:_TOPK]
    out_ref = row_and_out_refs[_TOPK]
    qi = pl.program_id(0)
    tv = tv_ref[pl.ds(qi, 1), :]
    m = jnp.max(tv)
    e = jnp.exp((tv - m) * 10.0)
    w = e / jnp.sum(e)
    acc = jnp.zeros((1, out_ref.shape[1]), jnp.float32)
    for j in range(_TOPK):
        r = idx_ref[qi * _TOPK + j] % 8
        row = row_refs[j][pl.ds(r, 1), :]
        acc = acc + w[:, j:j + 1] * row
    out_ref[pl.ds(qi, 1), :] = acc


def kernel(x, vault_keys):
    n_keys, d = vault_keys.shape
    q = x.shape[0]
    block = _BLOCK if n_keys % _BLOCK == 0 else 8192
    nb = math.ceil(n_keys / block)

    topv, topi = pl.pallas_call(
        functools.partial(_scan_body, block=block, n_keys=n_keys),
        grid=(nb,),
        in_specs=[
            pl.BlockSpec((q, d), lambda b: (0, 0)),
            pl.BlockSpec((block, d), lambda b: (b, 0)),
        ],
        out_specs=[
            pl.BlockSpec((q, _TOPK), lambda b: (0, 0)),
            pl.BlockSpec((q, _TOPK), lambda b: (0, 0)),
        ],
        out_shape=[
            jax.ShapeDtypeStruct((q, _TOPK), jnp.float32),
            jax.ShapeDtypeStruct((q, _TOPK), jnp.int32),
        ],
    )(x, vault_keys)

    idx_flat = topi.reshape(-1)

    def _row_spec(j):
        return pl.BlockSpec((8, d),
                            lambda qi, idx, j=j: (idx[qi * _TOPK + j] // 8, 0))

    out = pl.pallas_call(
        _readout_body,
        grid_spec=pltpu.PrefetchScalarGridSpec(
            num_scalar_prefetch=1,
            grid=(q,),
            in_specs=[pl.BlockSpec((q, _TOPK), lambda qi, idx: (0, 0))]
            + [_row_spec(j) for j in range(_TOPK)],
            out_specs=pl.BlockSpec((q, d), lambda qi, idx: (0, 0)),
        ),
        out_shape=jax.ShapeDtypeStruct((q, d), jnp.float32),
    )(idx_flat, topv, *([vault_keys] * _TOPK))

    return out


# speedup vs baseline: 1.1405x; 1.0931x over previous
import functools
import math

import jax
import jax.numpy as jnp
from jax.experimental import pallas as pl
from jax.experimental.pallas import tpu as pltpu

_BLOCK = 20000
_TOPK = 8
_NEG_INF = float("-inf")
_IMAX = jnp.iinfo(jnp.int32).max


def _group_sum4(v, bit0, bit1):
    d = v.shape[1]
    p1 = jnp.where(bit0, pltpu.roll(v, 1, axis=1),
                   pltpu.roll(v, d - 1, axis=1))
    s1 = v + p1
    p2 = jnp.where(bit1, pltpu.roll(s1, 2, axis=1),
                   pltpu.roll(s1, d - 2, axis=1))
    return s1 + p2


def _scan_body(x_ref, vk_ref, topv_ref, topi_ref, *, block, n_keys):
    b = pl.program_id(0)
    d = x_ref.shape[1]
    q = x_ref.shape[0]

    x = x_ref[...]
    lane_q = jax.lax.broadcasted_iota(jnp.int32, (q, d), 1)
    xn = jnp.sqrt(_group_sum4(x * x, lane_q % 2 == 1,
                              (lane_q // 2) % 2 == 1) + 1e-8)
    qn = (x / xn) * 0.5

    kb = vk_ref[...]
    g0 = jax.lax.broadcasted_iota(jnp.int32, (d, d), 0) // 4
    g1 = jax.lax.broadcasted_iota(jnp.int32, (d, d), 1) // 4
    mask = (g0 == g1).astype(jnp.float32)
    gs = jax.lax.dot_general(kb * kb, mask, (((1,), (0,)), ((), ())),
                             preferred_element_type=jnp.float32,
                             precision=jax.lax.Precision.HIGHEST)
    knb = kb / jnp.sqrt(gs + 1e-8)

    s = jax.lax.dot_general(qn, knb, (((1,), (1,)), ((), ())),
                            preferred_element_type=jnp.float32)

    gi = jax.lax.broadcasted_iota(jnp.int32, (q, block), 1) + b * block
    if n_keys % block != 0:
        s = jnp.where(gi < n_keys, s, _NEG_INF)
        gi = jnp.where(gi < n_keys, gi, _IMAX)

    @pl.when(b == 0)
    def _init():
        topv_ref[...] = jnp.full((q, _TOPK), _NEG_INF, jnp.float32)
        topi_ref[...] = jnp.full((q, _TOPK), _IMAX, jnp.int32)

    do_merge = jnp.any(jnp.max(s, axis=1) > topv_ref[:, _TOPK - 1])

    @pl.when(do_merge)
    def _merge():
        cs = jnp.concatenate([s, topv_ref[...]], axis=1)
        ci_all = jnp.concatenate([gi, topi_ref[...]], axis=1)

        lane = jax.lax.broadcasted_iota(jnp.int32, (q, _TOPK), 1)
        rv = jnp.full((q, _TOPK), _NEG_INF, jnp.float32)
        ri_out = jnp.full((q, _TOPK), _IMAX, jnp.int32)
        for j in range(_TOPK):
            m = jnp.max(cs, axis=1)
            sel = cs == m[:, None]
            cand = jnp.where(sel, ci_all, _IMAX)
            pick = jnp.min(cand, axis=1)
            chosen = sel & (ci_all == pick[:, None])
            rv = jnp.where(lane == j, m[:, None], rv)
            ri_out = jnp.where(lane == j, pick[:, None], ri_out)
            cs = jnp.where(chosen, _NEG_INF, cs)

        topv_ref[...] = rv
        topi_ref[...] = ri_out


def _readout_body(idx_ref, tv_ref, *row_and_out_refs):
    row_refs = row_and_out_refs[:_TOPK]
    out_ref = row_and_out_refs[_TOPK]
    qi = pl.program_id(0)
    tv = tv_ref[pl.ds(qi, 1), :]
    m = jnp.max(tv)
    e = jnp.exp((tv - m) * 10.0)
    w = e / jnp.sum(e)
    acc = jnp.zeros((1, out_ref.shape[1]), jnp.float32)
    for j in range(_TOPK):
        r = idx_ref[qi * _TOPK + j] % 8
        row = row_refs[j][pl.ds(r, 1), :]
        acc = acc + w[:, j:j + 1] * row
    out_ref[pl.ds(qi, 1), :] = acc


def kernel(x, vault_keys):
    n_keys, d = vault_keys.shape
    q = x.shape[0]
    block = _BLOCK if n_keys % _BLOCK == 0 else 8192
    nb = math.ceil(n_keys / block)

    topv, topi = pl.pallas_call(
        functools.partial(_scan_body, block=block, n_keys=n_keys),
        grid=(nb,),
        in_specs=[
            pl.BlockSpec((q, d), lambda b: (0, 0)),
            pl.BlockSpec((block, d), lambda b: (b, 0)),
        ],
        out_specs=[
            pl.BlockSpec((q, _TOPK), lambda b: (0, 0)),
            pl.BlockSpec((q, _TOPK), lambda b: (0, 0)),
        ],
        out_shape=[
            jax.ShapeDtypeStruct((q, _TOPK), jnp.float32),
            jax.ShapeDtypeStruct((q, _TOPK), jnp.int32),
        ],
    )(x, vault_keys)

    idx_flat = topi.reshape(-1)

    def _row_spec(j):
        return pl.BlockSpec((8, d),
                            lambda qi, idx, j=j: (idx[qi * _TOPK + j] // 8, 0))

    out = pl.pallas_call(
        _readout_body,
        grid_spec=pltpu.PrefetchScalarGridSpec(
            num_scalar_prefetch=1,
            grid=(q,),
            in_specs=[pl.BlockSpec((q, _TOPK), lambda qi, idx: (0, 0))]
            + [_row_spec(j) for j in range(_TOPK)],
            out_specs=pl.BlockSpec((q, d), lambda qi, idx: (0, 0)),
        ),
        out_shape=jax.ShapeDtypeStruct((q, d), jnp.float32),
    )(idx_flat, topv, *([vault_keys] * _TOPK))

    return out
